# baseline (speedup 1.0000x reference)
import jax
import jax.numpy as jnp
from jax.experimental import pallas as pl
from jax.experimental.pallas import tpu as pltpu

B, C, H, W = 8, 320, 48, 48
HW = H * W
LANES = 128
RPC = HW // LANES
CB = 64
RB = CB * RPC
NBLK = C // CB


def _scale_body(n_ref, g_ref, x_ref, o_ref):
    o_ref[...] = jnp.abs(g_ref[0]) * x_ref[...]


def kernel(x, n, gain_matrix):
    x3 = x.reshape(B, C * RPC, LANES)
    g_rep = jnp.broadcast_to(
        gain_matrix[:, :, None], (B, C, RPC)
    ).reshape(B, C * RPC, 1)
    out = pl.pallas_call(
        _scale_body,
        grid_spec=pltpu.PrefetchScalarGridSpec(
            num_scalar_prefetch=1,
            grid=(B, NBLK),
            in_specs=[
                pl.BlockSpec((1, RB, 1), lambda b, c, n_ref: (n_ref[b], c, 0)),
                pl.BlockSpec((1, RB, LANES), lambda b, c, n_ref: (b, c, 0)),
            ],
            out_specs=pl.BlockSpec((1, RB, LANES), lambda b, c, n_ref: (b, c, 0)),
        ),
        out_shape=jax.ShapeDtypeStruct((B, C * RPC, LANES), jnp.float32),
    )(n.astype(jnp.int32), g_rep, x3)
    return out.reshape(B, C, H, W)


# speedup vs baseline: 5.4499x; 5.4499x over previous
import jax
import jax.numpy as jnp
from jax.experimental import pallas as pl
from jax.experimental.pallas import tpu as pltpu

B, C, H, W = 8, 320, 48, 48
HW = H * W
CB = 40
NCHUNK_PER_B = C // CB
NCHUNKS = B * NCHUNK_PER_B
NBUF = 8


def _body(n_ref, gt_ref, x_ref, o_ref, ibuf, obuf, isem, osem):
    gta = jnp.abs(gt_ref[...])

    def chunk(s):
        return s // NCHUNK_PER_B, (s % NCHUNK_PER_B) * CB

    def start_in(s, slot):
        b, c0 = chunk(s)
        pltpu.make_async_copy(
            x_ref.at[b, pl.ds(c0, CB), :], ibuf.at[slot], isem.at[slot]
        ).start()

    for s in range(NBUF):
        start_in(s, s)

    g_col = None
    for s in range(NCHUNKS):
        slot = s % NBUF
        b, c0 = chunk(s)
        if c0 == 0:
            idx = n_ref[b]
            onehot = (
                jax.lax.broadcasted_iota(jnp.int32, (1, 8), 1) == idx
            ).astype(jnp.float32)
            g_col = jnp.sum(gta * onehot, axis=1, keepdims=True)
        pltpu.make_async_copy(
            x_ref.at[b, pl.ds(c0, CB), :], ibuf.at[slot], isem.at[slot]
        ).wait()
        if s >= NBUF:
            bp, cp0 = chunk(s - NBUF)
            pltpu.make_async_copy(
                obuf.at[slot], o_ref.at[bp, pl.ds(cp0, CB), :], osem.at[slot]
            ).wait()
        obuf[slot] = g_col[c0:c0 + CB] * ibuf[slot]
        pltpu.make_async_copy(
            obuf.at[slot], o_ref.at[b, pl.ds(c0, CB), :], osem.at[slot]
        ).start()
        if s + NBUF < NCHUNKS:
            start_in(s + NBUF, slot)

    for s in range(NCHUNKS - NBUF, NCHUNKS):
        slot = s % NBUF
        b, c0 = chunk(s)
        pltpu.make_async_copy(
            obuf.at[slot], o_ref.at[b, pl.ds(c0, CB), :], osem.at[slot]
        ).wait()


def kernel(x, n, gain_matrix):
    x3 = x.reshape(B, C, HW)
    gt = gain_matrix.T
    out = pl.pallas_call(
        _body,
        grid_spec=pltpu.PrefetchScalarGridSpec(
            num_scalar_prefetch=1,
            grid=(1,),
            in_specs=[
                pl.BlockSpec((C, 8), lambda i, n_ref: (0, 0)),
                pl.BlockSpec(memory_space=pl.ANY),
            ],
            out_specs=pl.BlockSpec(memory_space=pl.ANY),
            scratch_shapes=[
                pltpu.VMEM((NBUF, CB, HW), jnp.float32),
                pltpu.VMEM((NBUF, CB, HW), jnp.float32),
                pltpu.SemaphoreType.DMA((NBUF,)),
                pltpu.SemaphoreType.DMA((NBUF,)),
            ],
        ),
        out_shape=jax.ShapeDtypeStruct((B, C, HW), jnp.float32),
    )(n.astype(jnp.int32), gt, x3)
    return out.reshape(B, C, H, W)
